# Initial kernel scaffold; baseline (speedup 1.0000x reference)
import jax
import jax.numpy as jnp
from jax.experimental import pallas as pl


def kernel(x, edge_index, W, b):
    raise NotImplementedError("write your pallas kernel here")


# speedup vs baseline: 12.0676x; 12.0676x over previous
import functools

import jax
import jax.numpy as jnp
from jax import lax
from jax.experimental import pallas as pl
from jax.experimental.pallas import tpu as pltpu
from jax.experimental.pallas import tpu_sc as plsc

N = 10000
E = 320000
DIN = 128
DOUT = 64

NC = 2
NS = 16
NW = NC * NS
EPW = E // NW
B = 80
J = EPW // B
G = 5
RPT = N // NS
DEGW = 16

_mesh = plsc.VectorSubcoreMesh(core_axis_name="c", subcore_axis_name="s")
_sc_params = pltpu.CompilerParams(use_tc_tiling_on_sc=False)


@functools.partial(
    pl.kernel,
    out_type=jax.ShapeDtypeStruct((NC, N, DEGW), jnp.float32),
    mesh=_mesh,
    scratch_types=[
        pltpu.VMEM((J, B), jnp.int32),
        pltpu.VMEM((B, DEGW), jnp.float32),
        pltpu.VMEM((RPT, DEGW), jnp.float32),
        pltpu.VMEM_SHARED((N, DEGW), jnp.float32),
        pltpu.SemaphoreType.DMA,
    ],
    compiler_params=_sc_params,
)
def _deg_kernel(dst_hbm, degp_hbm, dstv, ones_v, zbuf, deg_sp, sem):
    c = lax.axis_index("c")
    s = lax.axis_index("s")
    wid = c * NS + s

    def _fill_z(i, carry):
        zbuf[i, :] = jnp.zeros((DEGW,), jnp.float32)
        return carry

    lax.fori_loop(0, RPT, _fill_z, 0)

    def _fill_o(i, carry):
        ones_v[i, :] = jnp.full((DEGW,), 1.0, jnp.float32)
        return carry

    lax.fori_loop(0, B, _fill_o, 0)

    pltpu.sync_copy(dst_hbm.at[wid], dstv)
    pltpu.sync_copy(zbuf, deg_sp.at[pl.ds(s * RPT, RPT)])
    plsc.subcore_barrier()

    def _group(gi, carry):
        descs = [
            pltpu.async_copy(ones_v, deg_sp.at[dstv.at[gi * G + t]], sem, add=True)
            for t in range(G)
        ]
        for d in descs:
            d.wait()
        return carry

    lax.fori_loop(0, J // G, _group, 0)
    plsc.subcore_barrier()

    pltpu.sync_copy(
        deg_sp.at[pl.ds(s * RPT, RPT)],
        degp_hbm.at[c, pl.ds(s * RPT, RPT)],
    )


def _g_body(x_ref, w_ref, degp_ref, g_ref):
    deg = degp_ref[0] + degp_ref[1] + 1.0
    dinv = lax.rsqrt(deg[:, 0:1])
    h0 = lax.dot_general(
        x_ref[...], w_ref[...], (((1,), (1,)), ((), ())),
        preferred_element_type=jnp.float32,
    )
    g_ref[...] = h0 * dinv


_g_call = pl.pallas_call(
    _g_body,
    out_shape=jax.ShapeDtypeStruct((N, DOUT), jnp.float32),
)


@functools.partial(
    pl.kernel,
    out_type=jax.ShapeDtypeStruct((NC, N, DOUT), jnp.float32),
    mesh=_mesh,
    scratch_types=[
        pltpu.VMEM((J, B), jnp.int32),
        pltpu.VMEM((J, B), jnp.int32),
        pltpu.VMEM((G, B, DOUT), jnp.float32),
        pltpu.VMEM((RPT // G, DOUT), jnp.float32),
        pltpu.VMEM_SHARED((N, DOUT), jnp.float32),
        pltpu.SemaphoreType.DMA,
        pltpu.SemaphoreType.DMA,
    ],
    compiler_params=_sc_params,
)
def _scatter_kernel(g_hbm, src_hbm, dst_hbm, accp_hbm,
                    srcv, dstv, rows, zbuf, acc_sp, gsem, ssem):
    c = lax.axis_index("c")
    s = lax.axis_index("s")
    wid = c * NS + s

    def _fill_z(i, carry):
        for q in range(DOUT // 16):
            zbuf[i, pl.ds(q * 16, 16)] = jnp.zeros((16,), jnp.float32)
        return carry

    lax.fori_loop(0, RPT // G, _fill_z, 0)

    pltpu.sync_copy(src_hbm.at[wid], srcv)
    pltpu.sync_copy(dst_hbm.at[wid], dstv)
    for q in range(G):
        pltpu.sync_copy(
            zbuf, acc_sp.at[pl.ds(s * RPT + q * (RPT // G), RPT // G)]
        )
    plsc.subcore_barrier()

    def _group(gi, carry):
        gd = [
            pltpu.async_copy(g_hbm.at[srcv.at[gi * G + t]], rows.at[t], gsem)
            for t in range(G)
        ]
        for d in gd:
            d.wait()
        sd = [
            pltpu.async_copy(rows.at[t], acc_sp.at[dstv.at[gi * G + t]],
                             ssem, add=True)
            for t in range(G)
        ]
        for d in sd:
            d.wait()
        return carry

    lax.fori_loop(0, J // G, _group, 0)
    plsc.subcore_barrier()

    pltpu.sync_copy(
        acc_sp.at[pl.ds(s * RPT, RPT)],
        accp_hbm.at[c, pl.ds(s * RPT, RPT)],
    )


def _h_body(accp_ref, g_ref, degp_ref, b_ref, h_ref):
    deg = degp_ref[0] + degp_ref[1] + 1.0
    dinv = lax.rsqrt(deg[:, 0:1])
    acc = accp_ref[0] + accp_ref[1] + g_ref[...]
    h_ref[...] = dinv * acc + b_ref[...]


_h_call = pl.pallas_call(
    _h_body,
    out_shape=jax.ShapeDtypeStruct((N, DOUT), jnp.float32),
)


@functools.partial(
    pl.kernel,
    out_type=(
        jax.ShapeDtypeStruct((E, DOUT), jnp.float32),
        jax.ShapeDtypeStruct((E, DOUT), jnp.float32),
    ),
    mesh=_mesh,
    scratch_types=[
        pltpu.VMEM((J, B), jnp.int32),
        pltpu.VMEM((J, B), jnp.int32),
        pltpu.VMEM((G, B, DOUT), jnp.float32),
        pltpu.VMEM((G, B, DOUT), jnp.float32),
        pltpu.SemaphoreType.DMA,
        pltpu.SemaphoreType.DMA,
    ],
    compiler_params=_sc_params,
)
def _pairs_kernel(h_hbm, src_hbm, dst_hbm, hs_hbm, hd_hbm,
                  srcv, dstv, rs, rd, gsem, wsem):
    c = lax.axis_index("c")
    s = lax.axis_index("s")
    wid = c * NS + s
    base = wid * EPW

    pltpu.sync_copy(src_hbm.at[wid], srcv)
    pltpu.sync_copy(dst_hbm.at[wid], dstv)

    def _group(gi, carry):
        gd = []
        for t in range(G):
            j = gi * G + t
            gd.append(pltpu.async_copy(h_hbm.at[srcv.at[j]], rs.at[t], gsem))
            gd.append(pltpu.async_copy(h_hbm.at[dstv.at[j]], rd.at[t], gsem))
        for d in gd:
            d.wait()
        wd = []
        for t in range(G):
            j = gi * G + t
            wd.append(pltpu.async_copy(
                rs.at[t], hs_hbm.at[pl.ds(base + j * B, B)], wsem))
            wd.append(pltpu.async_copy(
                rd.at[t], hd_hbm.at[pl.ds(base + j * B, B)], wsem))
        for d in wd:
            d.wait()
        return carry

    lax.fori_loop(0, J // G, _group, 0)


def _dot_body(hs_ref, hd_ref, o_ref):
    o_ref[...] = jnp.sum(hs_ref[...] * hd_ref[...], axis=1, keepdims=True)


_DOT_BLK = 20000
_dot_call = pl.pallas_call(
    _dot_body,
    grid=(E // _DOT_BLK,),
    in_specs=[
        pl.BlockSpec((_DOT_BLK, DOUT), lambda i: (i, 0)),
        pl.BlockSpec((_DOT_BLK, DOUT), lambda i: (i, 0)),
    ],
    out_specs=pl.BlockSpec((_DOT_BLK, 1), lambda i: (i, 0)),
    out_shape=jax.ShapeDtypeStruct((E, 1), jnp.float32),
)


def kernel(x, edge_index, W, b):
    src3 = edge_index[0].reshape(NW, J, B)
    dst3 = edge_index[1].reshape(NW, J, B)
    degp = _deg_kernel(dst3)
    g = _g_call(x, W, degp)
    accp = _scatter_kernel(g, src3, dst3)
    h = _h_call(accp, g, degp, b.reshape(1, DOUT))
    hs, hd = _pairs_kernel(h, src3, dst3)
    logits = _dot_call(hs, hd)
    return logits.reshape(E)
